# baseline (speedup 1.0000x reference)
import jax
import jax.numpy as jnp
from jax.experimental import pallas as pl

N = 50000
D = 128
_R = 1024


def _epilogue_body(agg_ref, cnt_ref, x_ref, wl_ref, bl_ref, wr_ref, wo_ref,
                   bo_ref, out_ref):
    agg = agg_ref[...]
    cnt = cnt_ref[...]
    x = x_ref[...]
    mean = agg * (1.0 / jnp.maximum(cnt, 1.0))
    pre = (jnp.dot(mean, wl_ref[...], preferred_element_type=jnp.float32)
           + jnp.dot(x, wr_ref[...], preferred_element_type=jnp.float32)
           + bl_ref[...])
    h = jnp.maximum(pre, 0.0)
    out_ref[...] = (jnp.dot(h, wo_ref[...], preferred_element_type=jnp.float32)
                    + bo_ref[...])


def _epilogue(agg, cnt, x, WlT, bl, WrT, WoT, bo):
    grid = (pl.cdiv(N, _R),)
    return pl.pallas_call(
        _epilogue_body,
        grid=grid,
        in_specs=[
            pl.BlockSpec((_R, D), lambda i: (i, 0)),
            pl.BlockSpec((_R, 1), lambda i: (i, 0)),
            pl.BlockSpec((_R, D), lambda i: (i, 0)),
            pl.BlockSpec((D, D), lambda i: (0, 0)),
            pl.BlockSpec((1, D), lambda i: (0, 0)),
            pl.BlockSpec((D, D), lambda i: (0, 0)),
            pl.BlockSpec((D, D), lambda i: (0, 0)),
            pl.BlockSpec((1, D), lambda i: (0, 0)),
        ],
        out_specs=pl.BlockSpec((_R, D), lambda i: (i, 0)),
        out_shape=jax.ShapeDtypeStruct((N, D), jnp.float32),
    )(agg, cnt, x, WlT, bl, WrT, WoT, bo)


def kernel(x_local, x_virtual, edge_index_ll, edge_index_lv, edge_index_vv,
           Wl_ll, bl_ll, Wr_ll, Wl_lv, bl_lv, Wr_lv, Wl_vv, bl_vv, Wr_vv,
           W_out, b_out):
    src = edge_index_ll[0]
    dst = edge_index_ll[1]
    msg = jnp.take(x_local, src, axis=0)
    agg = jax.ops.segment_sum(msg, dst, num_segments=N)
    cnt = jax.ops.segment_sum(jnp.ones((src.shape[0],), jnp.float32), dst,
                              num_segments=N)
    return _epilogue(agg, cnt[:, None], x_local, Wl_ll.T, bl_ll[None, :],
                     Wr_ll.T, W_out.T, b_out[None, :])
